# Initial kernel scaffold; baseline (speedup 1.0000x reference)
import jax
import jax.numpy as jnp
from jax.experimental import pallas as pl


def kernel(edge_index, in_feat1, in_feat2, W1, b1, W2, b2, W3, b3):
    raise NotImplementedError("write your pallas kernel here")


# speedup vs baseline: 3.6877x; 3.6877x over previous
import functools

import jax
import jax.numpy as jnp
from jax import lax
from jax.experimental import pallas as pl
from jax.experimental.pallas import tpu as pltpu
from jax.experimental.pallas import tpu_sc as plsc

N = 10000
E = 160000
SUB = 16
G = 128
GPW = 79
EPAD = SUB * GPW * G
NPAD = 10112
ZROWS = NPAD // SUB
LAST_ROWS = N - 15 * ZROWS
MB = 1000


def _sc_mesh():
    return plsc.VectorSubcoreMesh(core_axis_name="c", subcore_axis_name="s")


def _zero_slice(sid):
    return pl.ds(pl.multiple_of(sid * ZROWS, 8), ZROWS)


def _flush(acc_s, out_ref, sid):
    @pl.when(sid < SUB - 1)
    def _():
        s = pl.ds(pl.multiple_of(sid * ZROWS, 8), ZROWS)
        pltpu.sync_copy(acc_s.at[s], out_ref.at[s])

    @pl.when(sid == SUB - 1)
    def _():
        s = pl.ds(15 * ZROWS, LAST_ROWS)
        pltpu.sync_copy(acc_s.at[s], out_ref.at[s])


def _degrees(srcd3, dst3, ones_hbm, zeros_hbm):
    @functools.partial(
        pl.kernel,
        out_type=[jax.ShapeDtypeStruct((N, 128), jnp.float32)] * 2,
        mesh=_sc_mesh(),
        scratch_types=[
            pltpu.VMEM((GPW, G), jnp.int32),
            pltpu.VMEM((G, 128), jnp.float32),
            pltpu.VMEM_SHARED((NPAD, 128), jnp.float32),
            pltpu.SemaphoreType.DMA,
        ],
    )
    def deg_k(srcd_hbm, dst_hbm, ones_ref, zeros_ref, dego_hbm, degi_hbm,
              idx_v, ones_v, hist_s, sem):
        ci = lax.axis_index("c")
        sid = lax.axis_index("s")
        pltpu.sync_copy(zeros_ref, hist_s.at[_zero_slice(sid)])
        pltpu.sync_copy(ones_ref, ones_v)
        plsc.subcore_barrier()
        for core, (src_ref, out_ref) in enumerate(
            ((srcd_hbm, dego_hbm), (dst_hbm, degi_hbm))):
            @pl.when(ci == core)
            def _(src_ref=src_ref, out_ref=out_ref):
                pltpu.sync_copy(src_ref.at[sid], idx_v)

                @pl.loop(0, GPW)
                def _(g):
                    pltpu.sync_copy(ones_v, hist_s.at[idx_v.at[g]], add=True)

                plsc.subcore_barrier()
                _flush(hist_s, out_ref, sid)

    return deg_k(srcd3, dst3, ones_hbm, zeros_hbm)


def _aggregate(tables, srcg3, dst3, zeros_hbm, fc, edge_split=False):
    nch = len(tables)
    n_out = 2 if edge_split else nch

    @functools.partial(
        pl.kernel,
        out_type=[jax.ShapeDtypeStruct((N, fc), jnp.float32)] * n_out,
        mesh=_sc_mesh(),
        scratch_types=[
            pltpu.VMEM((GPW, G), jnp.int32),
            pltpu.VMEM((GPW, G), jnp.int32),
            pltpu.VMEM((G, fc), jnp.float32),
            pltpu.VMEM_SHARED((NPAD, fc), jnp.float32),
            pltpu.SemaphoreType.DMA,
        ],
    )
    def agg_k(*refs):
        srcg_hbm, dst_hbm, zeros_ref = refs[:3]
        tbls = refs[3:3 + nch]
        outs = refs[3 + nch:3 + nch + n_out]
        src_v, dst_v, buf_v, acc_s, sem = refs[3 + nch + n_out:]
        ci = lax.axis_index("c")
        sid = lax.axis_index("s")
        pltpu.sync_copy(srcg_hbm.at[sid], src_v)
        pltpu.sync_copy(dst_hbm.at[sid], dst_v)
        for core in range(2):
            if edge_split:
                work = [(tbls[0], outs[core],
                         (GPW // 2) * core, (GPW // 2) * (core + 1) + (GPW % 2) * core)]
            else:
                work = [(tbls[core + 2 * k], outs[core + 2 * k], 0, GPW)
                        for k in range(nch // 2)]

            @pl.when(ci == core)
            def _(work=work):
                for tbl, out, g0, g1 in work:
                    pltpu.sync_copy(zeros_ref, acc_s.at[_zero_slice(sid)])
                    plsc.subcore_barrier()

                    @pl.loop(g0, g1)
                    def _(g):
                        pltpu.async_copy(tbl.at[src_v.at[g]], buf_v, sem).wait()
                        pltpu.sync_copy(buf_v, acc_s.at[dst_v.at[g]], add=True)

                    plsc.subcore_barrier()
                    _flush(acc_s, out, sid)
                    plsc.subcore_barrier()

    return agg_k(srcg3, dst3, zeros_hbm, *tables)


def _norm(deg_block):
    return lax.rsqrt(jnp.maximum(deg_block[:, 0:1], 1.0))


def _mm1(f1, f2, w1a, w1b, dego):
    def body(f1_ref, f2_ref, dego_ref, w1a_ref, w1b_ref, o0, o1, o2, o3):
        h = jnp.dot(f1_ref[...], w1a_ref[...], preferred_element_type=jnp.float32)
        h = h + jnp.dot(f2_ref[...], w1b_ref[...], preferred_element_type=jnp.float32)
        h = h * _norm(dego_ref[...])
        for j, o in enumerate((o0, o1, o2, o3)):
            o[...] = h[:, j * 128:(j + 1) * 128]

    return pl.pallas_call(
        body,
        grid=(N // MB,),
        in_specs=[
            pl.BlockSpec((MB, 256), lambda i: (i, 0)),
            pl.BlockSpec((MB, 256), lambda i: (i, 0)),
            pl.BlockSpec((MB, 128), lambda i: (i, 0)),
            pl.BlockSpec((256, 512), lambda i: (0, 0)),
            pl.BlockSpec((256, 512), lambda i: (0, 0)),
        ],
        out_specs=[pl.BlockSpec((MB, 128), lambda i: (i, 0))] * 4,
        out_shape=[jax.ShapeDtypeStruct((N, 128), jnp.float32)] * 4,
    )(f1, f2, dego, w1a, w1b)


def _mm_mid(aggs, degi, dego, w, b, n_out_chunks, fc_out):
    f_in = 128 * len(aggs)

    def body(*refs):
        agg_refs = refs[:len(aggs)]
        degi_ref, dego_ref, w_ref, b_ref = refs[len(aggs):len(aggs) + 4]
        outs = refs[len(aggs) + 4:]
        x = jnp.concatenate([r[...] for r in agg_refs], axis=1)
        x = jax.nn.relu(x * _norm(degi_ref[...]) + b_ref[...])
        h = jnp.dot(x, w_ref[...], preferred_element_type=jnp.float32)
        h = h * _norm(dego_ref[...])
        if h.shape[1] < n_out_chunks * fc_out:
            zpad = jnp.zeros((h.shape[0], n_out_chunks * fc_out - h.shape[1]),
                             jnp.float32)
            h = jnp.concatenate([h, zpad], axis=1)
        for j, o in enumerate(outs):
            o[...] = h[:, j * fc_out:(j + 1) * fc_out]

    return pl.pallas_call(
        body,
        grid=(N // MB,),
        in_specs=(
            [pl.BlockSpec((MB, 128), lambda i: (i, 0))] * len(aggs)
            + [pl.BlockSpec((MB, 128), lambda i: (i, 0))] * 2
            + [pl.BlockSpec(w.shape, lambda i: (0, 0)),
               pl.BlockSpec((1, f_in), lambda i: (0, 0))]
        ),
        out_specs=[pl.BlockSpec((MB, fc_out), lambda i: (i, 0))] * n_out_chunks,
        out_shape=[jax.ShapeDtypeStruct((N, fc_out), jnp.float32)] * n_out_chunks,
    )(*aggs, degi, dego, w, b)


def _final_max(d0, d1, degi, b3):
    def body(d0_ref, d1_ref, degi_ref, b3_ref, out_ref):
        i = pl.program_id(0)
        y = (d0_ref[...] + d1_ref[...])[:, :64]
        y = y * _norm(degi_ref[...]) + b3_ref[...]
        m = jnp.max(y)

        @pl.when(i == 0)
        def _():
            out_ref[...] = jnp.full((1, 128), -jnp.inf, jnp.float32)

        out_ref[...] = jnp.maximum(out_ref[...], m)

    return pl.pallas_call(
        body,
        grid=(N // MB,),
        in_specs=[
            pl.BlockSpec((MB, 128), lambda i: (i, 0)),
            pl.BlockSpec((MB, 128), lambda i: (i, 0)),
            pl.BlockSpec((MB, 128), lambda i: (i, 0)),
            pl.BlockSpec((1, 64), lambda i: (0, 0)),
        ],
        out_specs=pl.BlockSpec((1, 128), lambda i: (0, 0)),
        out_shape=jax.ShapeDtypeStruct((1, 128), jnp.float32),
    )(d0, d1, degi, b3)


def kernel(edge_index, in_feat1, in_feat2, W1, b1, W2, b2, W3, b3):
    src = edge_index[0].astype(jnp.int32)
    dst = edge_index[1].astype(jnp.int32)
    pad = EPAD - E
    srcg3 = jnp.concatenate([src, jnp.zeros((pad,), jnp.int32)]).reshape(SUB, GPW, G)
    srcd3 = jnp.concatenate([src, jnp.full((pad,), N, jnp.int32)]).reshape(SUB, GPW, G)
    dst3 = jnp.concatenate([dst, jnp.full((pad,), N, jnp.int32)]).reshape(SUB, GPW, G)

    ones128 = jnp.ones((G, 128), jnp.float32)
    z128 = jnp.zeros((ZROWS, 128), jnp.float32)

    dego, degi = _degrees(srcd3, dst3, ones128, z128)

    h1 = _mm1(in_feat1, in_feat2, W1[:256], W1[256:], dego)
    a1 = _aggregate(h1, srcg3, dst3, z128, 128)

    h2 = _mm_mid(a1, degi, dego, W2, b1.reshape(1, 512), 2, 128)
    a2 = _aggregate(h2, srcg3, dst3, z128, 128)

    h3 = _mm_mid(a2, degi, dego, W3, b2.reshape(1, 256), 1, 128)
    a3 = _aggregate(h3, srcg3, dst3, z128, 128, edge_split=True)

    out = _final_max(a3[0], a3[1], degi, b3.reshape(1, 64))
    return out[0, 0]
